# speedup vs baseline: 5.2282x; 5.2282x over previous
import functools

import jax
import jax.numpy as jnp
from jax import lax
from jax.experimental import pallas as pl
from jax.experimental.pallas import tpu as pltpu
from jax.experimental.pallas import tpu_sc as plsc

D_MODEL = 128
HALF = 64
_NW = 32
_CHUNK = 128


def _elem_body(x_ref, w_ref, o_ref):
    w1 = w_ref[:, 0:HALF]
    o_ref[:] = lax.dot_general(x_ref[:], w1, (((1,), (1,)), ((), ())),
                               preferred_element_type=jnp.float32)


def _prop_body(x_ref, w_ref, b_ref, o_ref):
    w2 = w_ref[:, HALF:D_MODEL]
    o_ref[:] = lax.dot_general(x_ref[:], w2, (((1,), (1,)), ((), ())),
                               preferred_element_type=jnp.float32) + b_ref[:]


def _transform_elem(table, w):
    v = table.shape[0]
    r = 2000
    return pl.pallas_call(
        _elem_body,
        grid=(v // r,),
        in_specs=[pl.BlockSpec((r, HALF), lambda i: (i, 0)),
                  pl.BlockSpec((D_MODEL, D_MODEL), lambda i: (0, 0))],
        out_specs=pl.BlockSpec((r, D_MODEL), lambda i: (i, 0)),
        out_shape=jax.ShapeDtypeStruct((v, D_MODEL), jnp.float32),
    )(table, w)


def _transform_prop(table, w, b2d):
    v = table.shape[0]
    return pl.pallas_call(
        _prop_body,
        in_specs=[pl.BlockSpec((v, HALF), lambda: (0, 0)),
                  pl.BlockSpec((D_MODEL, D_MODEL), lambda: (0, 0)),
                  pl.BlockSpec((1, D_MODEL), lambda: (0, 0))],
        out_specs=pl.BlockSpec((v, D_MODEL), lambda: (0, 0)),
        out_shape=jax.ShapeDtypeStruct((v, D_MODEL), jnp.float32),
    )(table, w, b2d)


def _sc_combine(te, tp, eid, pid):
    b = eid.shape[0]
    per_w = b // _NW
    n_iter = per_w // _CHUNK
    mesh = plsc.VectorSubcoreMesh(core_axis_name="c", subcore_axis_name="s")

    @functools.partial(
        pl.kernel, mesh=mesh,
        out_type=jax.ShapeDtypeStruct((b, D_MODEL), jnp.float32),
        scratch_types=[
            pltpu.VMEM((_CHUNK,), jnp.int32),
            pltpu.VMEM((_CHUNK,), jnp.int32),
            pltpu.VMEM((_CHUNK, D_MODEL), jnp.float32),
            pltpu.VMEM((_CHUNK, D_MODEL), jnp.float32),
            pltpu.SemaphoreType.DMA,
        ],
    )
    def k(te_hbm, tp_hbm, eid_hbm, pid_hbm, out_hbm, eidx_v, pidx_v, ebuf,
          pbuf, sem):
        wid = lax.axis_index("s") * 2 + lax.axis_index("c")
        base = wid * per_w

        def body(it, carry):
            off = base + it * _CHUNK
            pltpu.sync_copy(eid_hbm.at[pl.ds(off, _CHUNK)], eidx_v)
            pltpu.sync_copy(pid_hbm.at[pl.ds(off, _CHUNK)], pidx_v)
            pltpu.async_copy(te_hbm.at[eidx_v], ebuf, sem).wait()
            pltpu.async_copy(tp_hbm.at[pidx_v], pbuf, sem).wait()

            def addrow(r2, c2):
                for j in range(D_MODEL // 16):
                    sl = pl.ds(j * 16, 16)
                    ebuf[r2, sl] = ebuf[r2, sl] + pbuf[r2, sl]
                return c2

            lax.fori_loop(0, _CHUNK, addrow, 0)
            pltpu.sync_copy(ebuf, out_hbm.at[pl.ds(off, _CHUNK)])
            return carry

        lax.fori_loop(0, n_iter, body, 0)

    return k(te, tp, eid, pid)


def kernel(element_ids, property_ids, element_table, property_table,
           fusion_w, fusion_b):
    bsz, n = element_ids.shape
    eid = element_ids.reshape(-1).astype(jnp.int32)
    pid = property_ids.reshape(-1).astype(jnp.int32)
    te = _transform_elem(element_table, fusion_w)
    tp = _transform_prop(property_table, fusion_w, fusion_b.reshape(1, -1))
    out = _sc_combine(te, tp, eid, pid)
    return out.reshape(bsz, n, D_MODEL)


# speedup vs baseline: 6.0328x; 1.1539x over previous
import functools

import jax
import jax.numpy as jnp
from jax import lax
from jax.experimental import pallas as pl
from jax.experimental.pallas import tpu as pltpu
from jax.experimental.pallas import tpu_sc as plsc

D_MODEL = 128
HALF = 64
_NW = 32
_CHUNK = 128


def _elem_body(x_ref, w_ref, o_ref):
    w1 = w_ref[:, 0:HALF]
    o_ref[:] = lax.dot_general(x_ref[:], w1, (((1,), (1,)), ((), ())),
                               preferred_element_type=jnp.float32)


def _prop_body(x_ref, w_ref, b_ref, o_ref):
    w2 = w_ref[:, HALF:D_MODEL]
    o_ref[:] = lax.dot_general(x_ref[:], w2, (((1,), (1,)), ((), ())),
                               preferred_element_type=jnp.float32) + b_ref[:]


def _transform_elem(table, w):
    v = table.shape[0]
    r = 2000
    return pl.pallas_call(
        _elem_body,
        grid=(v // r,),
        in_specs=[pl.BlockSpec((r, HALF), lambda i: (i, 0)),
                  pl.BlockSpec((D_MODEL, D_MODEL), lambda i: (0, 0))],
        out_specs=pl.BlockSpec((r, D_MODEL), lambda i: (i, 0)),
        out_shape=jax.ShapeDtypeStruct((v, D_MODEL), jnp.float32),
    )(table, w)


def _transform_prop(table, w, b2d):
    v = table.shape[0]
    return pl.pallas_call(
        _prop_body,
        in_specs=[pl.BlockSpec((v, HALF), lambda: (0, 0)),
                  pl.BlockSpec((D_MODEL, D_MODEL), lambda: (0, 0)),
                  pl.BlockSpec((1, D_MODEL), lambda: (0, 0))],
        out_specs=pl.BlockSpec((v, D_MODEL), lambda: (0, 0)),
        out_shape=jax.ShapeDtypeStruct((v, D_MODEL), jnp.float32),
    )(table, w, b2d)


def _sc_combine(te, tp, eid, pid):
    b = eid.shape[0]
    per_w = b // _NW
    n_iter = per_w // _CHUNK
    mesh = plsc.VectorSubcoreMesh(core_axis_name="c", subcore_axis_name="s")

    @functools.partial(
        pl.kernel, mesh=mesh,
        out_type=jax.ShapeDtypeStruct((b, D_MODEL), jnp.float32),
        scratch_types=[
            pltpu.VMEM((_CHUNK,), jnp.int32),
            pltpu.VMEM((_CHUNK,), jnp.int32),
            pltpu.VMEM((_CHUNK, D_MODEL), jnp.float32),
            pltpu.VMEM((_CHUNK, D_MODEL), jnp.float32),
            pltpu.VMEM_SHARED((1000, D_MODEL), jnp.float32),
            pltpu.SemaphoreType.DMA,
        ],
    )
    def k(te_hbm, tp_hbm, eid_hbm, pid_hbm, out_hbm, eidx_v, pidx_v, ebuf,
          pbuf, tp_sh, sem):
        wid = lax.axis_index("s") * 2 + lax.axis_index("c")
        base = wid * per_w

        @pl.when(lax.axis_index("s") == 0)
        def _stage():
            pltpu.sync_copy(tp_hbm, tp_sh)

        plsc.subcore_barrier()

        def body(it, carry):
            off = base + it * _CHUNK
            pltpu.sync_copy(eid_hbm.at[pl.ds(off, _CHUNK)], eidx_v)
            pltpu.sync_copy(pid_hbm.at[pl.ds(off, _CHUNK)], pidx_v)
            pltpu.async_copy(te_hbm.at[eidx_v], ebuf, sem).wait()
            pltpu.async_copy(tp_sh.at[pidx_v], pbuf, sem).wait()

            def addrow(r2, c2):
                for j in range(D_MODEL // 16):
                    sl = pl.ds(j * 16, 16)
                    ebuf[r2, sl] = ebuf[r2, sl] + pbuf[r2, sl]
                return c2

            lax.fori_loop(0, _CHUNK, addrow, 0)
            pltpu.sync_copy(ebuf, out_hbm.at[pl.ds(off, _CHUNK)])
            return carry

        lax.fori_loop(0, n_iter, body, 0)

    return k(te, tp, eid, pid)


def kernel(element_ids, property_ids, element_table, property_table,
           fusion_w, fusion_b):
    bsz, n = element_ids.shape
    eid = element_ids.reshape(-1).astype(jnp.int32)
    pid = property_ids.reshape(-1).astype(jnp.int32)
    te = _transform_elem(element_table, fusion_w)
    tp = _transform_prop(property_table, fusion_w, fusion_b.reshape(1, -1))
    out = _sc_combine(te, tp, eid, pid)
    return out.reshape(bsz, n, D_MODEL)


# speedup vs baseline: 14.7601x; 2.4467x over previous
import functools

import jax
import jax.numpy as jnp
from jax import lax
from jax.experimental import pallas as pl
from jax.experimental.pallas import tpu as pltpu
from jax.experimental.pallas import tpu_sc as plsc

D_MODEL = 128
HALF = 64
_NW = 32
_CHUNK = 128


def _elem_body(x_ref, w_ref, o_ref):
    w1 = w_ref[:, 0:HALF]
    o_ref[:] = lax.dot_general(x_ref[:], w1, (((1,), (1,)), ((), ())),
                               preferred_element_type=jnp.float32)


def _prop_body(x_ref, w_ref, b_ref, o_ref):
    w2 = w_ref[:, HALF:D_MODEL]
    o_ref[:] = lax.dot_general(x_ref[:], w2, (((1,), (1,)), ((), ())),
                               preferred_element_type=jnp.float32) + b_ref[:]


def _transform_elem(table, w):
    v = table.shape[0]
    r = 2000
    return pl.pallas_call(
        _elem_body,
        grid=(v // r,),
        in_specs=[pl.BlockSpec((r, HALF), lambda i: (i, 0)),
                  pl.BlockSpec((D_MODEL, D_MODEL), lambda i: (0, 0))],
        out_specs=pl.BlockSpec((r, D_MODEL), lambda i: (i, 0)),
        out_shape=jax.ShapeDtypeStruct((v, D_MODEL), jnp.float32),
    )(table, w)


def _transform_prop(table, w, b2d):
    v = table.shape[0]
    return pl.pallas_call(
        _prop_body,
        in_specs=[pl.BlockSpec((v, HALF), lambda: (0, 0)),
                  pl.BlockSpec((D_MODEL, D_MODEL), lambda: (0, 0)),
                  pl.BlockSpec((1, D_MODEL), lambda: (0, 0))],
        out_specs=pl.BlockSpec((v, D_MODEL), lambda: (0, 0)),
        out_shape=jax.ShapeDtypeStruct((v, D_MODEL), jnp.float32),
    )(table, w, b2d)


def _sc_combine(te, tp, eid, pid):
    b = eid.shape[0]
    per_w = b // _NW
    n_iter = per_w // _CHUNK
    mesh = plsc.VectorSubcoreMesh(core_axis_name="c", subcore_axis_name="s")

    assert n_iter % 4 == 0 and n_iter >= 8

    @functools.partial(
        pl.kernel, mesh=mesh,
        out_type=jax.ShapeDtypeStruct((b, D_MODEL), jnp.float32),
        scratch_types=[
            pltpu.VMEM((4, _CHUNK), jnp.int32),
            pltpu.VMEM((4, _CHUNK), jnp.int32),
            pltpu.VMEM((2, _CHUNK, D_MODEL), jnp.float32),
            pltpu.VMEM((2, _CHUNK, D_MODEL), jnp.float32),
            pltpu.VMEM((2, _CHUNK, D_MODEL), jnp.float32),
            pltpu.VMEM_SHARED((1000, D_MODEL), jnp.float32),
            pltpu.SemaphoreType.DMA((4,)),
            pltpu.SemaphoreType.DMA((2,)),
            pltpu.SemaphoreType.DMA((2,)),
            pltpu.SemaphoreType.DMA((2,)),
        ],
    )
    def k(te_hbm, tp_hbm, eid_hbm, pid_hbm, out_hbm, eixb, pixb, ebufs,
          pbufs, obufs, tp_sh, isem, esem, psem, wsem):
        wid = lax.axis_index("s") * 2 + lax.axis_index("c")
        base = wid * per_w

        @pl.when(lax.axis_index("s") == 0)
        def _stage():
            pltpu.sync_copy(tp_hbm, tp_sh)

        plsc.subcore_barrier()

        for i in (0, 1):
            off = base + i * _CHUNK
            pltpu.sync_copy(eid_hbm.at[pl.ds(off, _CHUNK)], eixb.at[i])
            pltpu.sync_copy(pid_hbm.at[pl.ds(off, _CHUNK)], pixb.at[i])
        for i in (0, 1):
            pltpu.async_copy(te_hbm.at[eixb.at[i]], ebufs.at[i], esem.at[i])
            pltpu.async_copy(tp_sh.at[pixb.at[i]], pbufs.at[i], psem.at[i])
        for i in (2, 3):
            off = base + i * _CHUNK
            pltpu.async_copy(eid_hbm.at[pl.ds(off, _CHUNK)], eixb.at[i],
                             isem.at[i])
            pltpu.async_copy(pid_hbm.at[pl.ds(off, _CHUNK)], pixb.at[i],
                             isem.at[i])

        def quad(fq, carry):
            for sub in range(4):
                i = fq * 4 + sub
                bb = sub % 2
                qq = sub

                pltpu.make_async_copy(te_hbm.at[eixb.at[qq]], ebufs.at[bb],
                                      esem.at[bb]).wait()
                pltpu.make_async_copy(tp_sh.at[pixb.at[qq]], pbufs.at[bb],
                                      psem.at[bb]).wait()

                @pl.when(i + 4 < n_iter)
                def _pfidx():
                    off4 = base + (i + 4) * _CHUNK
                    pltpu.async_copy(eid_hbm.at[pl.ds(off4, _CHUNK)],
                                     eixb.at[qq], isem.at[qq])
                    pltpu.async_copy(pid_hbm.at[pl.ds(off4, _CHUNK)],
                                     pixb.at[qq], isem.at[qq])

                @pl.when(i >= 2)
                def _wdrain():
                    pltpu.make_async_copy(
                        obufs.at[bb], out_hbm.at[pl.ds(base, _CHUNK)],
                        wsem.at[bb]).wait()

                def addrow(r2, c2):
                    for j in range(D_MODEL // 16):
                        sl = pl.ds(j * 16, 16)
                        obufs[bb, r2, sl] = ebufs[bb, r2, sl] + pbufs[bb, r2, sl]
                    return c2

                lax.fori_loop(0, _CHUNK, addrow, 0)

                off = base + i * _CHUNK
                pltpu.async_copy(obufs.at[bb], out_hbm.at[pl.ds(off, _CHUNK)],
                                 wsem.at[bb])

                @pl.when(i + 2 < n_iter)
                def _gnext():
                    q2 = (sub + 2) % 4
                    pltpu.make_async_copy(
                        eid_hbm.at[pl.ds(base, _CHUNK)], eixb.at[q2],
                        isem.at[q2]).wait()
                    pltpu.make_async_copy(
                        pid_hbm.at[pl.ds(base, _CHUNK)], pixb.at[q2],
                        isem.at[q2]).wait()
                    pltpu.async_copy(te_hbm.at[eixb.at[q2]], ebufs.at[bb],
                                     esem.at[bb])
                    pltpu.async_copy(tp_sh.at[pixb.at[q2]], pbufs.at[bb],
                                     psem.at[bb])
            return carry

        lax.fori_loop(0, n_iter // 4, quad, 0)

        for bb in (0, 1):
            pltpu.make_async_copy(obufs.at[bb],
                                  out_hbm.at[pl.ds(base, _CHUNK)],
                                  wsem.at[bb]).wait()

    return k(te, tp, eid, pid)


def kernel(element_ids, property_ids, element_table, property_table,
           fusion_w, fusion_b):
    bsz, n = element_ids.shape
    eid = element_ids.reshape(-1).astype(jnp.int32)
    pid = property_ids.reshape(-1).astype(jnp.int32)
    te = _transform_elem(element_table, fusion_w)
    tp = _transform_prop(property_table, fusion_w, fusion_b.reshape(1, -1))
    out = _sc_combine(te, tp, eid, pid)
    return out.reshape(bsz, n, D_MODEL)


# speedup vs baseline: 14.7685x; 1.0006x over previous
import functools

import jax
import jax.numpy as jnp
from jax import lax
from jax.experimental import pallas as pl
from jax.experimental.pallas import tpu as pltpu
from jax.experimental.pallas import tpu_sc as plsc

D_MODEL = 128
HALF = 64
_NW = 32
_CHUNK = 128


def _elem_body(x_ref, w_ref, o_ref):
    w1 = w_ref[:, 0:HALF]
    o_ref[:] = lax.dot_general(x_ref[:], w1, (((1,), (1,)), ((), ())),
                               preferred_element_type=jnp.float32)


def _prop_body(x_ref, w_ref, b_ref, o_ref):
    w2 = w_ref[:, HALF:D_MODEL]
    o_ref[:] = lax.dot_general(x_ref[:], w2, (((1,), (1,)), ((), ())),
                               preferred_element_type=jnp.float32) + b_ref[:]


def _transform_elem(table, w):
    v = table.shape[0]
    r = 2000
    return pl.pallas_call(
        _elem_body,
        grid=(v // r,),
        in_specs=[pl.BlockSpec((r, HALF), lambda i: (i, 0)),
                  pl.BlockSpec((D_MODEL, D_MODEL), lambda i: (0, 0))],
        out_specs=pl.BlockSpec((r, D_MODEL), lambda i: (i, 0)),
        out_shape=jax.ShapeDtypeStruct((v, D_MODEL), jnp.float32),
    )(table, w)


def _transform_prop(table, w, b2d):
    v = table.shape[0]
    return pl.pallas_call(
        _prop_body,
        in_specs=[pl.BlockSpec((v, HALF), lambda: (0, 0)),
                  pl.BlockSpec((D_MODEL, D_MODEL), lambda: (0, 0)),
                  pl.BlockSpec((1, D_MODEL), lambda: (0, 0))],
        out_specs=pl.BlockSpec((v, D_MODEL), lambda: (0, 0)),
        out_shape=jax.ShapeDtypeStruct((v, D_MODEL), jnp.float32),
    )(table, w, b2d)


def _sc_combine(te, tp, eid, pid):
    b = eid.shape[0]
    per_w = b // _NW
    n_iter = per_w // _CHUNK
    mesh = plsc.VectorSubcoreMesh(core_axis_name="c", subcore_axis_name="s")

    assert n_iter % 4 == 0 and n_iter >= 8

    @functools.partial(
        pl.kernel, mesh=mesh,
        out_type=jax.ShapeDtypeStruct((b, D_MODEL), jnp.float32),
        scratch_types=[
            pltpu.VMEM((4, _CHUNK), jnp.int32),
            pltpu.VMEM((4, _CHUNK), jnp.int32),
            pltpu.VMEM((2, _CHUNK, D_MODEL), jnp.float32),
            pltpu.VMEM((2, _CHUNK, D_MODEL), jnp.float32),
            pltpu.VMEM((2, _CHUNK, D_MODEL), jnp.float32),
            pltpu.VMEM_SHARED((1000, D_MODEL), jnp.float32),
            pltpu.SemaphoreType.DMA((4,)),
            pltpu.SemaphoreType.DMA((2,)),
            pltpu.SemaphoreType.DMA((2,)),
            pltpu.SemaphoreType.DMA((2,)),
        ],
    )
    def k(te_hbm, tp_hbm, eid_hbm, pid_hbm, out_hbm, eixb, pixb, ebufs,
          pbufs, obufs, tp_sh, isem, esem, psem, wsem):
        wid = lax.axis_index("s") * 2 + lax.axis_index("c")
        base = wid * per_w

        @pl.when(lax.axis_index("s") == 0)
        def _stage():
            pltpu.sync_copy(tp_hbm, tp_sh)

        plsc.subcore_barrier()

        for i in (0, 1):
            off = base + i * _CHUNK
            pltpu.sync_copy(eid_hbm.at[pl.ds(off, _CHUNK)], eixb.at[i])
            pltpu.sync_copy(pid_hbm.at[pl.ds(off, _CHUNK)], pixb.at[i])
        for i in (0, 1):
            pltpu.async_copy(te_hbm.at[eixb.at[i]], ebufs.at[i], esem.at[i])
            pltpu.async_copy(tp_sh.at[pixb.at[i]], pbufs.at[i], psem.at[i])
        for i in (2, 3):
            off = base + i * _CHUNK
            pltpu.async_copy(eid_hbm.at[pl.ds(off, _CHUNK)], eixb.at[i],
                             isem.at[i])
            pltpu.async_copy(pid_hbm.at[pl.ds(off, _CHUNK)], pixb.at[i],
                             isem.at[i])

        def quad(fq, carry):
            for sub in range(4):
                i = fq * 4 + sub
                bb = sub % 2
                qq = sub

                pltpu.make_async_copy(te_hbm.at[eixb.at[qq]], ebufs.at[bb],
                                      esem.at[bb]).wait()
                pltpu.make_async_copy(tp_sh.at[pixb.at[qq]], pbufs.at[bb],
                                      psem.at[bb]).wait()

                @pl.when(i + 4 < n_iter)
                def _pfidx():
                    off4 = base + (i + 4) * _CHUNK
                    pltpu.async_copy(eid_hbm.at[pl.ds(off4, _CHUNK)],
                                     eixb.at[qq], isem.at[qq])
                    pltpu.async_copy(pid_hbm.at[pl.ds(off4, _CHUNK)],
                                     pixb.at[qq], isem.at[qq])

                @pl.when(i >= 2)
                def _wdrain():
                    pltpu.make_async_copy(
                        obufs.at[bb], out_hbm.at[pl.ds(base, _CHUNK)],
                        wsem.at[bb]).wait()

                def addrow(r4, c2):
                    for rr in range(4):
                        r2 = r4 * 4 + rr
                        for j in range(D_MODEL // 16):
                            sl = pl.ds(j * 16, 16)
                            obufs[bb, r2, sl] = (ebufs[bb, r2, sl]
                                                 + pbufs[bb, r2, sl])
                    return c2

                lax.fori_loop(0, _CHUNK // 4, addrow, 0)

                off = base + i * _CHUNK
                pltpu.async_copy(obufs.at[bb], out_hbm.at[pl.ds(off, _CHUNK)],
                                 wsem.at[bb])

                @pl.when(i + 2 < n_iter)
                def _gnext():
                    q2 = (sub + 2) % 4
                    pltpu.make_async_copy(
                        eid_hbm.at[pl.ds(base, _CHUNK)], eixb.at[q2],
                        isem.at[q2]).wait()
                    pltpu.make_async_copy(
                        pid_hbm.at[pl.ds(base, _CHUNK)], pixb.at[q2],
                        isem.at[q2]).wait()
                    pltpu.async_copy(te_hbm.at[eixb.at[q2]], ebufs.at[bb],
                                     esem.at[bb])
                    pltpu.async_copy(tp_sh.at[pixb.at[q2]], pbufs.at[bb],
                                     psem.at[bb])
            return carry

        lax.fori_loop(0, n_iter // 4, quad, 0)

        for bb in (0, 1):
            pltpu.make_async_copy(obufs.at[bb],
                                  out_hbm.at[pl.ds(base, _CHUNK)],
                                  wsem.at[bb]).wait()

    return k(te, tp, eid, pid)


def kernel(element_ids, property_ids, element_table, property_table,
           fusion_w, fusion_b):
    bsz, n = element_ids.shape
    eid = element_ids.reshape(-1).astype(jnp.int32)
    pid = property_ids.reshape(-1).astype(jnp.int32)
    te = _transform_elem(element_table, fusion_w)
    tp = _transform_prop(property_table, fusion_w, fusion_b.reshape(1, -1))
    out = _sc_combine(te, tp, eid, pid)
    return out.reshape(bsz, n, D_MODEL)
